# baseline (speedup 1.0000x reference)
import functools

import jax
import jax.numpy as jnp
from jax import lax
from jax.experimental import pallas as pl
from jax.experimental.pallas import tpu as pltpu
from jax.experimental.pallas import tpu_sc as plsc

VOCAB_N = 1000000
EMBED = 64
ROW = 128
B_TOT = 4096 * 200
NW = 32
B_PER_W = B_TOT // NW
CHUNK = 512
N_CHUNKS = B_PER_W // CHUNK
NBUF = 2
N_ROUNDS = N_CHUNKS // NBUF

BCOLS = 256
NBLK = VOCAB_N // BCOLS
TAIL_C0 = VOCAB_N - 128

_mesh = plsc.VectorSubcoreMesh(core_axis_name="c", subcore_axis_name="s")


@functools.partial(
    pl.kernel,
    mesh=_mesh,
    out_type=jax.ShapeDtypeStruct((VOCAB_N * EMBED,), jnp.float32),
    scratch_types=[pltpu.VMEM((EMBED, BCOLS), jnp.float32)] * 3
    + [pltpu.VMEM((BCOLS * EMBED,), jnp.float32)] * 3
    + [pltpu.SemaphoreType.DMA] * 6,
    compiler_params=pltpu.CompilerParams(
        use_tc_tiling_on_sc=True, needs_layout_passes=False
    ),
)
def _transpose_table(tt_hbm, tail_hbm, out_hbm, s0, s1, s2, d0, d1, d2, *sems):
    svmem = (s0, s1, s2)
    dvmem = (d0, d1, d2)
    i_sem = sems[0:3]
    o_sem = sems[3:6]

    wid = lax.axis_index("s") * 2 + lax.axis_index("c")

    iota = lax.iota(jnp.int32, 16)
    jslots = [(iota + k) & 15 for k in range(16)]

    def c0_of(t):
        blk = jnp.minimum(wid + NW * t, NBLK - 1)
        return pl.multiple_of(blk * BCOLS, BCOLS)

    def in_copy(t, b):
        return pltpu.make_async_copy(
            tt_hbm.at[:, pl.ds(c0_of(t), BCOLS)], svmem[b], i_sem[b]
        )

    def tail_in_copy(b):
        return pltpu.make_async_copy(
            tail_hbm.at[:, :], svmem[b].at[:, pl.ds(0, 128)], i_sem[b]
        )

    def out_copy(c0, b):
        return pltpu.make_async_copy(
            dvmem[b], out_hbm.at[pl.ds(c0 * EMBED, BCOLS * EMBED)], o_sem[b]
        )

    def tail_out_copy(b):
        return pltpu.make_async_copy(
            dvmem[b].at[pl.ds(0, 128 * EMBED)],
            out_hbm.at[pl.ds(TAIL_C0 * EMBED, 128 * EMBED)],
            o_sem[b],
        )

    def transpose_block(b, nid):
        s = svmem[b]
        d = dvmem[b]
        nrow = nid // 16

        @plsc.parallel_loop(0, nrow * 4, step=1, unroll=8)
        def _(sb):
            I = (sb % nrow) * 16
            J = (sb // nrow) * 16
            iv = I + iota
            dbase = iv * EMBED + J
            for k in range(16):
                jv = J + jslots[k]
                v = plsc.load_gather(s, [jv, iv])
                plsc.store_scatter(d, [dbase + jslots[k]], v)

    NT = -(-NBLK // NW)
    NTRI = NT // 3

    in_copy(0, 0).start()
    in_copy(1, 1).start()

    def body(r, carry):
        for bb in range(3):
            t = r * 3 + bb
            in_copy(t, bb).wait()

            @pl.when(t + 2 < NT)
            def _():
                in_copy(t + 2, (bb + 2) % 3).start()

            @pl.when(t >= 3)
            def _():
                out_copy(c0_of(t - 3), bb).wait()

            transpose_block(bb, BCOLS)
            out_copy(c0_of(t), bb).start()
        return carry

    lax.fori_loop(0, NTRI, body, 0)

    @pl.when(wid == NW - 1)
    def _():
        tail_in_copy(0).start()

    out_copy(c0_of(NT - 3), 0).wait()

    @pl.when(wid == NW - 1)
    def _():
        tail_in_copy(0).wait()
        transpose_block(0, 128)
        tail_out_copy(0).start()
        tail_out_copy(0).wait()

    out_copy(c0_of(NT - 2), 1).wait()
    out_copy(c0_of(NT - 1), 2).wait()


L_DIM = 200
NBH = 32
ITEM_N = L_DIM
OUT_WORDS = L_DIM * EMBED * 4096


@functools.partial(
    pl.kernel,
    mesh=_mesh,
    out_type=jax.ShapeDtypeStruct((OUT_WORDS,), jnp.float32),
    scratch_types=[pltpu.VMEM((128,), jnp.int32)] * 2
    + [pltpu.VMEM((128, EMBED), jnp.float32)] * 2
    + [pltpu.VMEM((EMBED * 128,), jnp.float32)] * 2
    + [pltpu.SemaphoreType.DMA] * 6,
    compiler_params=pltpu.CompilerParams(
        use_tc_tiling_on_sc=False, needs_layout_passes=False
    ),
)
def _embed_gather(table_hbm, idx_hbm, out_hbm, x0, x1, g0, g1, d0, d1, *sems):
    idx_v = (x0, x1)
    g_v = (g0, g1)
    d_v = (d0, d1)
    i_sem = sems[0:2]
    g_sem = sems[2:4]
    o_sem = sems[4:6]

    wid = lax.axis_index("s") * 2 + lax.axis_index("c")

    iota = lax.iota(jnp.int32, 16)
    jslots = [(iota + k) & 15 for k in range(16)]

    def idx_copy(t, b):
        return pltpu.make_async_copy(
            idx_hbm.at[pl.ds(t * 4096 + wid * 128, 128)], idx_v[b], i_sem[b]
        )

    def gather_copy(b):
        return pltpu.make_async_copy(table_hbm.at[idx_v[b]], g_v[b], g_sem[b])

    def store_copy(t, b, jh):
        return pltpu.make_async_copy(
            d_v[b].at[pl.ds(jh * 1024, 1024)],
            out_hbm.at[pl.ds(((t * 8 + jh) * NBH + wid) * 1024, 1024)],
            o_sem[b],
        )

    def transpose_item(b):
        g = g_v[b]
        d = d_v[b]

        @plsc.parallel_loop(0, 32, step=1, unroll=8)
        def _(sb):
            I = (sb % 8) * 16
            J = (sb // 8) * 16
            tok = I + iota
            for k in range(16):
                col = J + jslots[k]
                v = plsc.load_gather(g, [tok, col])
                plsc.store_scatter(d, [col * 128 + tok], v)

    idx_copy(0, 0).start()
    idx_copy(1, 1).start()
    idx_copy(0, 0).wait()
    gather_copy(0).start()

    def body(r, carry):
        for bb in range(2):
            t = r * 2 + bb
            gather_copy(bb).wait()

            @pl.when(t + 2 < ITEM_N)
            def _():
                idx_copy(t + 2, bb).start()

            @pl.when(t + 1 < ITEM_N)
            def _():
                idx_copy(t + 1, 1 - bb).wait()
                gather_copy(1 - bb).start()

            @pl.when(t >= 2)
            def _():
                for jh in range(8):
                    store_copy(t - 2, bb, jh).wait()

            transpose_item(bb)
            for jh in range(8):
                store_copy(t, bb, jh).start()
        return carry

    lax.fori_loop(0, ITEM_N // 2, body, 0)

    for bb in range(2):
        for jh in range(8):
            store_copy(ITEM_N - 2 + bb, bb, jh).wait()


def kernel(sentence, table):
    idx = jnp.transpose(sentence).reshape(-1).astype(jnp.int32)
    tt = jnp.transpose(table)
    tail = lax.slice(tt, (0, TAIL_C0), (EMBED, VOCAB_N))
    flat = _transpose_table(tt, tail)
    t64 = flat.reshape(VOCAB_N, EMBED)
    out = _embed_gather(t64, idx)
    x = out.reshape(L_DIM, 8, NBH, 8, 128)
    return x.transpose(2, 4, 0, 1, 3).reshape(4096, L_DIM, EMBED)


# speedup vs baseline: 1.4159x; 1.4159x over previous
import functools

import jax
import jax.numpy as jnp
from jax import lax
from jax.experimental import pallas as pl
from jax.experimental.pallas import tpu as pltpu
from jax.experimental.pallas import tpu_sc as plsc

VOCAB_N = 1000000
EMBED = 64
ROW = 128
B_TOT = 4096 * 200
NW = 32
B_PER_W = B_TOT // NW
CHUNK = 512
N_CHUNKS = B_PER_W // CHUNK
NBUF = 2
N_ROUNDS = N_CHUNKS // NBUF

BCOLS = 256
NBLK = VOCAB_N // BCOLS
TAIL_C0 = VOCAB_N - 128

_mesh = plsc.VectorSubcoreMesh(core_axis_name="c", subcore_axis_name="s")


@functools.partial(
    pl.kernel,
    mesh=_mesh,
    out_type=jax.ShapeDtypeStruct((VOCAB_N * EMBED,), jnp.float32),
    scratch_types=[pltpu.VMEM((EMBED, BCOLS), jnp.float32)] * 3
    + [pltpu.VMEM((BCOLS * EMBED,), jnp.float32)] * 3
    + [pltpu.SemaphoreType.DMA] * 6,
    compiler_params=pltpu.CompilerParams(
        use_tc_tiling_on_sc=True, needs_layout_passes=False
    ),
)
def _transpose_table(tt_hbm, tail_hbm, out_hbm, s0, s1, s2, d0, d1, d2, *sems):
    svmem = (s0, s1, s2)
    dvmem = (d0, d1, d2)
    i_sem = sems[0:3]
    o_sem = sems[3:6]

    wid = lax.axis_index("s") * 2 + lax.axis_index("c")

    iota = lax.iota(jnp.int32, 16)
    jslots = [(iota + k) & 15 for k in range(16)]

    def c0_of(t):
        blk = jnp.minimum(wid + NW * t, NBLK - 1)
        return pl.multiple_of(blk * BCOLS, BCOLS)

    def in_copy(t, b):
        return pltpu.make_async_copy(
            tt_hbm.at[:, pl.ds(c0_of(t), BCOLS)], svmem[b], i_sem[b]
        )

    def tail_in_copy(b):
        return pltpu.make_async_copy(
            tail_hbm.at[:, :], svmem[b].at[:, pl.ds(0, 128)], i_sem[b]
        )

    def out_copy(c0, b):
        return pltpu.make_async_copy(
            dvmem[b], out_hbm.at[pl.ds(c0 * EMBED, BCOLS * EMBED)], o_sem[b]
        )

    def tail_out_copy(b):
        return pltpu.make_async_copy(
            dvmem[b].at[pl.ds(0, 128 * EMBED)],
            out_hbm.at[pl.ds(TAIL_C0 * EMBED, 128 * EMBED)],
            o_sem[b],
        )

    def transpose_block(b, nid):
        s = svmem[b]
        d = dvmem[b]
        nrow = nid // 16

        @plsc.parallel_loop(0, nrow * 4, step=1, unroll=8)
        def _(sb):
            I = (sb % nrow) * 16
            J = (sb // nrow) * 16
            iv = I + iota
            dbase = iv * EMBED + J
            for k in range(16):
                jv = J + jslots[k]
                v = plsc.load_gather(s, [jv, iv])
                plsc.store_scatter(d, [dbase + jslots[k]], v)

    NT = -(-NBLK // NW)
    NTRI = NT // 3

    in_copy(0, 0).start()
    in_copy(1, 1).start()

    def body(r, carry):
        for bb in range(3):
            t = r * 3 + bb
            in_copy(t, bb).wait()

            @pl.when(t + 2 < NT)
            def _():
                in_copy(t + 2, (bb + 2) % 3).start()

            @pl.when(t >= 3)
            def _():
                out_copy(c0_of(t - 3), bb).wait()

            transpose_block(bb, BCOLS)
            out_copy(c0_of(t), bb).start()
        return carry

    lax.fori_loop(0, NTRI, body, 0)

    @pl.when(wid == NW - 1)
    def _():
        tail_in_copy(0).start()

    out_copy(c0_of(NT - 3), 0).wait()

    @pl.when(wid == NW - 1)
    def _():
        tail_in_copy(0).wait()
        transpose_block(0, 128)
        tail_out_copy(0).start()
        tail_out_copy(0).wait()

    out_copy(c0_of(NT - 2), 1).wait()
    out_copy(c0_of(NT - 1), 2).wait()


L_DIM = 200
NBH = 32
ITEM_N = L_DIM // 2
OUT_WORDS = L_DIM * EMBED * 4096


@functools.partial(
    pl.kernel,
    mesh=_mesh,
    out_type=jax.ShapeDtypeStruct((OUT_WORDS,), jnp.float32),
    scratch_types=[pltpu.VMEM((256,), jnp.int32)] * 2
    + [pltpu.VMEM((256, EMBED), jnp.float32)] * 2
    + [pltpu.VMEM((2 * EMBED * 128,), jnp.float32)] * 2
    + [pltpu.SemaphoreType.DMA] * 6,
    compiler_params=pltpu.CompilerParams(
        use_tc_tiling_on_sc=False, needs_layout_passes=False
    ),
)
def _embed_gather(table_hbm, idx_hbm, out_hbm, x0, x1, g0, g1, d0, d1, *sems):
    idx_v = (x0, x1)
    g_v = (g0, g1)
    d_v = (d0, d1)
    i_sem = sems[0:2]
    g_sem = sems[2:4]
    o_sem = sems[4:6]

    wid = lax.axis_index("s") * 2 + lax.axis_index("c")

    iota = lax.iota(jnp.int32, 16)
    jslots = [(iota + k) & 15 for k in range(16)]

    def idx_copies(t, b):
        return [
            pltpu.make_async_copy(
                idx_hbm.at[pl.ds((2 * t + li) * 4096 + wid * 128, 128)],
                idx_v[b].at[pl.ds(li * 128, 128)],
                i_sem[b],
            )
            for li in range(2)
        ]

    def gather_copy(b):
        return pltpu.make_async_copy(table_hbm.at[idx_v[b]], g_v[b], g_sem[b])

    def store_copy(t, b, li, jh):
        return pltpu.make_async_copy(
            d_v[b].at[pl.ds(li * 8192 + jh * 1024, 1024)],
            out_hbm.at[pl.ds((((2 * t + li) * 8 + jh) * NBH + wid) * 1024, 1024)],
            o_sem[b],
        )

    def transpose_item(b):
        g = g_v[b]
        d = d_v[b]

        @plsc.parallel_loop(0, 64, step=1, unroll=8)
        def _(sb):
            li = sb & 1
            s2 = sb >> 1
            I = (s2 % 8) * 16
            J = (s2 // 8) * 16
            tok = li * 128 + I + iota
            dbase = li * 8192 + I + iota
            for k in range(16):
                col = J + jslots[k]
                v = plsc.load_gather(g, [tok, col])
                plsc.store_scatter(d, [dbase + col * 128], v)

    for c in idx_copies(0, 0):
        c.start()
    for c in idx_copies(1, 1):
        c.start()
    for c in idx_copies(0, 0):
        c.wait()
    gather_copy(0).start()

    def body(r, carry):
        for bb in range(2):
            t = r * 2 + bb
            gather_copy(bb).wait()

            @pl.when(t + 2 < ITEM_N)
            def _():
                for c in idx_copies(t + 2, bb):
                    c.start()

            @pl.when(t + 1 < ITEM_N)
            def _():
                for c in idx_copies(t + 1, 1 - bb):
                    c.wait()
                gather_copy(1 - bb).start()

            @pl.when(t >= 2)
            def _():
                for li in range(2):
                    for jh in range(8):
                        store_copy(t - 2, bb, li, jh).wait()

            transpose_item(bb)
            for li in range(2):
                for jh in range(8):
                    store_copy(t, bb, li, jh).start()
        return carry

    lax.fori_loop(0, ITEM_N // 2, body, 0)

    for bb in range(2):
        for li in range(2):
            for jh in range(8):
                store_copy(ITEM_N - 2 + bb, bb, li, jh).wait()
def kernel(sentence, table):
    idx = jnp.transpose(sentence).reshape(-1).astype(jnp.int32)
    tt = jnp.transpose(table)
    tail = lax.slice(tt, (0, TAIL_C0), (EMBED, VOCAB_N))
    flat = _transpose_table(tt, tail)
    t64 = flat.reshape(VOCAB_N, EMBED)
    out = _embed_gather(t64, idx)
    x = out.reshape(L_DIM, 8, NBH, 8, 128)
    return x.transpose(2, 4, 0, 1, 3).reshape(4096, L_DIM, EMBED)
